# baseline (speedup 1.0000x reference)
import functools

import jax
import jax.numpy as jnp
from jax import lax
from jax.experimental import pallas as pl
from jax.experimental.pallas import tpu as pltpu
from jax.experimental.pallas import tpu_sc as plsc

_NC = 2
_NS = 16
_NW = _NC * _NS
_L = 16


def _sc_body(rows_per_w, ch, d, x_hbm, emb_hbm, idx_hbm, out_hbm,
             idx_v, row_v, in0, in1, ou0, ou1, si0, si1, so0, so1):
    wid = lax.axis_index("s") * _NC + lax.axis_index("c")
    base = wid * rows_per_w

    pltpu.sync_copy(idx_hbm, idx_v)
    i = idx_v[...][0]
    pltpu.sync_copy(emb_hbm.at[i], row_v)

    nch = rows_per_w // ch
    nj = d // _L
    ins = (in0, in1)
    ous = (ou0, ou1)
    sis = (si0, si1)
    sos = (so0, so1)

    def start_in(k, slot):
        pltpu.async_copy(x_hbm.at[pl.ds(base + k * ch, ch)], ins[slot],
                         sis[slot])

    def wait_in(slot):
        pltpu.make_async_copy(x_hbm.at[pl.ds(base, ch)], ins[slot],
                              sis[slot]).wait()

    def start_out(k, slot):
        pltpu.async_copy(ous[slot], out_hbm.at[pl.ds(base + k * ch, ch)],
                         sos[slot])

    def wait_out(slot):
        pltpu.make_async_copy(ous[slot], out_hbm.at[pl.ds(base, ch)],
                              sos[slot]).wait()

    start_in(0, 0)

    def step(k, b):
        @pl.when(k + 1 < nch)
        def _():
            start_in(k + 1, 1 - b)
        wait_in(b)

        @pl.when(k >= 2)
        def _():
            wait_out(b)

        def rowfn(r, c2):
            for j in range(nj):
                sl = pl.ds(j * _L, _L)
                ous[b][r, sl] = ins[b][r, sl] + row_v[sl]
            return c2

        lax.fori_loop(0, ch, rowfn, 0)
        start_out(k, b)

    def pair(g, carry):
        step(2 * g, 0)
        step(2 * g + 1, 1)
        return carry

    lax.fori_loop(0, nch // 2, pair, 0)
    wait_out(0)
    wait_out(1)


def kernel(x, type_emb, index):
    B, S, D = x.shape
    N = B * S
    assert D % _L == 0
    xf = x.reshape(N, D)
    idx = jnp.broadcast_to(jnp.asarray(index, jnp.int32), (_L,))

    rows_per_w = N // _NW
    ch = 16

    mesh = plsc.VectorSubcoreMesh(core_axis_name="c", subcore_axis_name="s")
    body = functools.partial(_sc_body, rows_per_w, ch, D)
    out = pl.kernel(
        body,
        out_type=jax.ShapeDtypeStruct((N, D), x.dtype),
        mesh=mesh,
        scratch_types=[
            pltpu.VMEM((_L,), jnp.int32),
            pltpu.VMEM((D,), jnp.float32),
            pltpu.VMEM((ch, D), jnp.float32),
            pltpu.VMEM((ch, D), jnp.float32),
            pltpu.VMEM((ch, D), jnp.float32),
            pltpu.VMEM((ch, D), jnp.float32),
            pltpu.SemaphoreType.DMA,
            pltpu.SemaphoreType.DMA,
            pltpu.SemaphoreType.DMA,
            pltpu.SemaphoreType.DMA,
        ],
    )(xf, type_emb, idx)
    return out.reshape(B, S, D)


# speedup vs baseline: 4.2399x; 4.2399x over previous
import jax
import jax.numpy as jnp
from jax.experimental import pallas as pl
from jax.experimental.pallas import tpu as pltpu


def _body(idx_ref, x_ref, emb_ref, o_ref):
    i = idx_ref[0]
    row = emb_ref[pl.ds(i, 1), :]
    o_ref[...] = x_ref[...] + row


def kernel(x, type_emb, index):
    B, S, D = x.shape
    N = B * S
    xf = x.reshape(N, D)
    idx = jnp.asarray(index, jnp.int32).reshape(1)

    BM = 2048
    grid = (N // BM,)

    out = pl.pallas_call(
        _body,
        grid_spec=pltpu.PrefetchScalarGridSpec(
            num_scalar_prefetch=1,
            grid=grid,
            in_specs=[
                pl.BlockSpec((BM, D), lambda i, s: (i, 0)),
                pl.BlockSpec((2, D), lambda i, s: (0, 0)),
            ],
            out_specs=pl.BlockSpec((BM, D), lambda i, s: (i, 0)),
        ),
        out_shape=jax.ShapeDtypeStruct((N, D), x.dtype),
    )(idx, xf, type_emb)
    return out.reshape(B, S, D)
